# baseline (speedup 1.0000x reference)
import functools

import jax
import jax.numpy as jnp
from jax import lax
from jax.experimental import pallas as pl
from jax.experimental.pallas import tpu as pltpu
from jax.experimental.pallas import tpu_sc as plsc

NC = 2
NS = 16
NW = NC * NS
GPC = 2
NBUF = 4


@functools.partial(jax.jit, static_argnums=(2,))
def _embed(idx, weight, d):
    n_chunks, chunk = idx.shape[1], idx.shape[2] * idx.shape[3]
    glen = chunk // GPC
    assert n_chunks % NBUF == 0 and idx.shape[2] == GPC
    n_rounds = n_chunks // NBUF
    mesh = plsc.VectorSubcoreMesh(core_axis_name="c", subcore_axis_name="s")

    @functools.partial(
        pl.kernel,
        mesh=mesh,
        out_type=jax.ShapeDtypeStruct((NW * n_chunks * chunk, d), jnp.float32),
        scratch_types=(
            [pltpu.VMEM((n_chunks, GPC, glen), jnp.int32)]
            + [pltpu.VMEM((chunk, d), jnp.float32) for _ in range(NBUF)]
            + [pltpu.SemaphoreType.DMA for _ in range(2 * NBUF)]
        ),
        compiler_params=pltpu.CompilerParams(use_tc_tiling_on_sc=True),
    )
    def emb(idx_hbm, table_hbm, out_hbm, idx_v, *bufs_and_sems):
        bufs = bufs_and_sems[:NBUF]
        gsem = bufs_and_sems[NBUF:2 * NBUF]
        wsem = bufs_and_sems[2 * NBUF:]
        wid = lax.axis_index("s") * NC + lax.axis_index("c")
        pltpu.sync_copy(idx_hbm.at[wid], idx_v)

        def gather(c, b):
            for g in range(GPC):
                pltpu.async_copy(table_hbm.at[idx_v.at[c, g]],
                                 bufs[b].at[pl.ds(g * glen, glen)], gsem[b])

        def wait_gather(c, b):
            for g in range(GPC):
                pltpu.make_async_copy(
                    table_hbm.at[idx_v.at[c, g]],
                    bufs[b].at[pl.ds(g * glen, glen)], gsem[b]).wait()

        for b in range(NBUF):
            gather(b, b)

        def round_body(r, carry):
            for b in range(NBUF):
                c = r * NBUF + b
                wait_gather(c, b)
                row = (wid * n_chunks + c) * chunk
                pltpu.async_copy(
                    bufs[b], out_hbm.at[pl.ds(row, chunk)], wsem[b])
                pltpu.make_async_copy(
                    bufs[b], out_hbm.at[pl.ds(row, chunk)], wsem[b]).wait()

                @pl.when(r + 1 < n_rounds)
                def _():
                    gather(c + NBUF, b)
            return carry

        lax.fori_loop(0, n_rounds, round_body, 0)

    return emb(idx, weight)


def kernel(x, weight):
    b0, s = x.shape
    v, d = weight.shape
    glen = 2 * s
    chunk = GPC * glen
    assert (b0 * s) % (NW * chunk) == 0
    n_chunks = b0 * s // (NW * chunk)
    idx = x.astype(jnp.int32).reshape(NW, n_chunks, GPC, glen)
    out = _embed(idx, weight, d)
    return out.reshape(b0, s, d)


# speedup vs baseline: 1.7770x; 1.7770x over previous
import functools

import jax
import jax.numpy as jnp
from jax import lax
from jax.experimental import pallas as pl
from jax.experimental.pallas import tpu as pltpu
from jax.experimental.pallas import tpu_sc as plsc

NC = 2
NS = 16
NW = NC * NS
GPC = 4
ROWS = 4
NBUF = 4


@functools.partial(jax.jit, static_argnums=(2, 3))
def _embed(idx, weight, s, d):
    n_chunks = idx.shape[1]
    glen = idx.shape[3]
    assert n_chunks % NBUF == 0 and idx.shape[2] == GPC and glen == s
    n_rounds = n_chunks // NBUF
    mesh = plsc.VectorSubcoreMesh(core_axis_name="c", subcore_axis_name="s")

    @functools.partial(
        pl.kernel,
        mesh=mesh,
        out_type=jax.ShapeDtypeStruct((NW * n_chunks * ROWS, s, d),
                                      jnp.float32),
        scratch_types=(
            [pltpu.VMEM((n_chunks, GPC, glen), jnp.int32)]
            + [pltpu.VMEM((ROWS, s, d), jnp.float32) for _ in range(NBUF)]
            + [pltpu.SemaphoreType.DMA for _ in range(2 * NBUF)]
        ),
        compiler_params=pltpu.CompilerParams(use_tc_tiling_on_sc=True),
    )
    def emb(idx_hbm, table_hbm, out_hbm, idx_v, *bufs_and_sems):
        bufs = bufs_and_sems[:NBUF]
        gsem = bufs_and_sems[NBUF:2 * NBUF]
        wsem = bufs_and_sems[2 * NBUF:]
        wid = lax.axis_index("s") * NC + lax.axis_index("c")
        pltpu.sync_copy(idx_hbm.at[wid], idx_v)

        def gather(c, b):
            for g in range(GPC):
                pltpu.async_copy(table_hbm.at[idx_v.at[c, g]],
                                 bufs[b].at[g], gsem[b])

        def wait_gather(c, b):
            for g in range(GPC):
                pltpu.make_async_copy(
                    table_hbm.at[idx_v.at[c, g]],
                    bufs[b].at[g], gsem[b]).wait()

        for b in range(NBUF):
            gather(b, b)

        def round_body(r, carry):
            for b in range(NBUF):
                c = r * NBUF + b
                wait_gather(c, b)
                row = (wid * n_chunks + c) * ROWS
                pltpu.async_copy(
                    bufs[b], out_hbm.at[pl.ds(row, ROWS)], wsem[b])
                pltpu.make_async_copy(
                    bufs[b], out_hbm.at[pl.ds(row, ROWS)], wsem[b]).wait()

                @pl.when(r + 1 < n_rounds)
                def _():
                    gather(c + NBUF, b)
            return carry

        lax.fori_loop(0, n_rounds, round_body, 0)

    return emb(idx, weight)


def kernel(x, weight):
    b0, s = x.shape
    v, d = weight.shape
    glen = s
    assert (b0 * s) % (NW * GPC * glen) == 0
    n_chunks = b0 * s // (NW * GPC * glen)
    idx = x.astype(jnp.int32).reshape(NW, n_chunks, GPC, glen)
    out = _embed(idx, weight, s, d)
    return out.reshape(b0, s, d)


# speedup vs baseline: 1.7891x; 1.0068x over previous
import functools

import jax
import jax.numpy as jnp
from jax import lax
from jax.experimental import pallas as pl
from jax.experimental.pallas import tpu as pltpu
from jax.experimental.pallas import tpu_sc as plsc

NC = 2
NS = 16
NW = NC * NS
GPC = 2
ROWS = 2
NBUF = 8


@functools.partial(jax.jit, static_argnums=(2, 3))
def _embed(idx, weight, s, d):
    n_chunks = idx.shape[1]
    glen = idx.shape[3]
    assert n_chunks % NBUF == 0 and idx.shape[2] == GPC and glen == s
    n_rounds = n_chunks // NBUF
    mesh = plsc.VectorSubcoreMesh(core_axis_name="c", subcore_axis_name="s")

    @functools.partial(
        pl.kernel,
        mesh=mesh,
        out_type=jax.ShapeDtypeStruct((NW * n_chunks * ROWS, s, d),
                                      jnp.float32),
        scratch_types=(
            [pltpu.VMEM((n_chunks, GPC, glen), jnp.int32)]
            + [pltpu.VMEM((ROWS, s, d), jnp.float32) for _ in range(NBUF)]
            + [pltpu.SemaphoreType.DMA for _ in range(2 * NBUF)]
        ),
        compiler_params=pltpu.CompilerParams(use_tc_tiling_on_sc=True),
    )
    def emb(idx_hbm, table_hbm, out_hbm, idx_v, *bufs_and_sems):
        bufs = bufs_and_sems[:NBUF]
        gsem = bufs_and_sems[NBUF:2 * NBUF]
        wsem = bufs_and_sems[2 * NBUF:]
        wid = lax.axis_index("s") * NC + lax.axis_index("c")
        pltpu.sync_copy(idx_hbm.at[wid], idx_v)

        def gather(c, b):
            for g in range(GPC):
                pltpu.async_copy(table_hbm.at[idx_v.at[c, g]],
                                 bufs[b].at[g], gsem[b])

        def wait_gather(c, b):
            for g in range(GPC):
                pltpu.make_async_copy(
                    table_hbm.at[idx_v.at[c, g]],
                    bufs[b].at[g], gsem[b]).wait()

        for b in range(NBUF):
            gather(b, b)

        def round_body(r, carry):
            for b in range(NBUF):
                c = r * NBUF + b
                wait_gather(c, b)
                row = (wid * n_chunks + c) * ROWS
                pltpu.async_copy(
                    bufs[b], out_hbm.at[pl.ds(row, ROWS)], wsem[b])
                pltpu.make_async_copy(
                    bufs[b], out_hbm.at[pl.ds(row, ROWS)], wsem[b]).wait()

                @pl.when(r + 1 < n_rounds)
                def _():
                    gather(c + NBUF, b)
            return carry

        lax.fori_loop(0, n_rounds, round_body, 0)

    return emb(idx, weight)


def kernel(x, weight):
    b0, s = x.shape
    v, d = weight.shape
    glen = s
    assert (b0 * s) % (NW * GPC * glen) == 0
    n_chunks = b0 * s // (NW * GPC * glen)
    idx = x.astype(jnp.int32).reshape(NW, n_chunks, GPC, glen)
    out = _embed(idx, weight, s, d)
    return out.reshape(b0, s, d)
